# baseline (speedup 1.0000x reference)
import jax
import jax.numpy as jnp
from jax import lax
from jax.experimental import pallas as pl
from jax.experimental.pallas import tpu as pltpu

_E = 64
_BLK = 1024
_N = 32768
_GRID = _N // _BLK


def _moe_body(x_ref, w_ref, out_ref, cnt_ref, psum_ref, nv_ref):
    i = pl.program_id(0)

    @pl.when(i == 0)
    def _init():
        cnt_ref[...] = jnp.zeros_like(cnt_ref)
        psum_ref[...] = jnp.zeros_like(psum_ref)
        nv_ref[0, 0] = jnp.float32(0.0)

    xb = x_ref[...]
    w = w_ref[0]
    m = jnp.max(xb, axis=1, keepdims=True)
    iota = lax.broadcasted_iota(jnp.int32, (_BLK, _E), 1)
    elig = xb == m
    first = jnp.min(jnp.where(elig, iota, _E), axis=1, keepdims=True)
    onehot = (iota == first).astype(jnp.float32)

    cnt_ref[...] += jnp.dot(w, onehot, preferred_element_type=jnp.float32)
    psum_ref[...] += jnp.dot(w, xb * onehot, preferred_element_type=jnp.float32)
    nv_ref[0, 0] += jnp.sum(w)

    @pl.when(i == _GRID - 1)
    def _fin():
        nv = nv_ref[0, 0]
        loss = _E * jnp.sum(cnt_ref[...] * psum_ref[...]) / (nv * nv * nv)
        out_ref[...] = jnp.full((1, 1), loss, dtype=jnp.float32)


def kernel(x, mask):
    xr = x.reshape(_N, _E)
    w = (mask == 1).astype(jnp.float32).reshape(_GRID, 1, _BLK)
    out = pl.pallas_call(
        _moe_body,
        grid=(_GRID,),
        in_specs=[
            pl.BlockSpec((_BLK, _E), lambda i: (i, 0)),
            pl.BlockSpec((1, 1, _BLK), lambda i: (i, 0, 0)),
        ],
        out_specs=pl.BlockSpec((1, 1), lambda i: (0, 0)),
        out_shape=jax.ShapeDtypeStruct((1, 1), jnp.float32),
        scratch_shapes=[
            pltpu.VMEM((1, _E), jnp.float32),
            pltpu.VMEM((1, _E), jnp.float32),
            pltpu.SMEM((1, 1), jnp.float32),
        ],
    )(xr, w)
    return out[0, 0]
